# baseline (speedup 1.0000x reference)
import jax
import jax.numpy as jnp
from jax import lax
from jax.experimental import pallas as pl
from jax.experimental.pallas import tpu as pltpu
from jax.experimental.pallas import tpu_sc as plsc

NR = 4096
NL = 8192
NB = 4
NW = 32
RPW = NR // NW
NCH = NL // 16
RG = 6
SHIFT = 13
BIG = 0x7FFFE000
IMAX = 0x7FFFFFFF
TOPK = 10
NPL = 4

_GDN = lax.GatherDimensionNumbers(
    offset_dims=(), collapsed_slice_dims=(0,), start_index_map=(0,))


def _vgather(x, idxv):
    return lax.gather(x, idxv[:, None], _GDN, (1,),
                      mode=lax.GatherScatterMode.PROMISE_IN_BOUNDS)


def _splat(val, dtype=jnp.int32):
    return jnp.full((16,), val, dtype)


def _align16(x):
    return (x + 15) & ~15


def _sel4(b, vals):
    return jnp.where(b == 0, vals[0],
                     jnp.where(b == 1, vals[1],
                               jnp.where(b == 2, vals[2], vals[3])))


def _sc_body(lx_h, ly_h, lz_h, lb_h, rx_h, ry_h, rz_h, rb_h, p3_h, a_h, lft_h,
             matched_h, part_h,
             lxv, lyv, lzv, lbv, rxv, ryv, rzv, rbv, p3v, av, bestv, nnbuf,
             plv, accb, mbuf, pbuf, lcol, rlist, sem):
    cid = lax.axis_index("c")
    sid = lax.axis_index("s")
    wid = sid * 2 + cid
    base = wid * RPW

    cps = [
        pltpu.async_copy(lx_h, lxv, sem),
        pltpu.async_copy(ly_h, lyv, sem),
        pltpu.async_copy(lz_h, lzv, sem),
        pltpu.async_copy(lb_h, lbv, sem),
        pltpu.async_copy(rx_h.at[pl.ds(base, RPW)], rxv, sem),
        pltpu.async_copy(ry_h.at[pl.ds(base, RPW)], ryv, sem),
        pltpu.async_copy(rz_h.at[pl.ds(base, RPW)], rzv, sem),
        pltpu.async_copy(rb_h.at[pl.ds(base, RPW)], rbv, sem),
        pltpu.async_copy(p3_h.at[pl.ds(base, RPW)], p3v, sem),
        pltpu.async_copy(a_h.at[pl.ds(base, RPW)], av, sem),
    ]
    for cp in cps:
        cp.wait()

    iota = lax.iota(jnp.int32, 16)
    imaxv = _splat(IMAX)

    def lcnt_body(c, cnts):
        lbc = lbv[pl.ds(c * 16, 16)]
        return tuple(cnts[b] + jnp.sum((lbc == b).astype(jnp.int32))
                     for b in range(NB))

    lcnt = lax.fori_loop(0, NCH, lcnt_body, (0, 0, 0, 0))
    ls1 = _align16(lcnt[0])
    ls2 = ls1 + _align16(lcnt[1])
    ls3 = ls2 + _align16(lcnt[2])
    lstarts = (0, ls1, ls2, ls3)

    def lwr_body(c, offs):
        lbc = lbv[pl.ds(c * 16, 16)]
        colv = iota + c * 16
        out = []
        for b in range(NB):
            mask = lbc == b
            plsc.store_compressed(lcol.at[pl.ds(offs[b], 16)], colv, mask=mask)
            out.append(offs[b] + jnp.sum(mask.astype(jnp.int32)))
        return tuple(out)

    lax.fori_loop(0, NCH, lwr_body, lstarts)

    def rcnt_body(c, cnts):
        rbc = rbv[pl.ds(c * 16, 16)]
        return tuple(cnts[b] + jnp.sum((rbc == b).astype(jnp.int32))
                     for b in range(NB))

    rcnt = lax.fori_loop(0, RPW // 16, rcnt_body, (0, 0, 0, 0))
    rs1 = _align16(rcnt[0])
    rs2 = rs1 + _align16(rcnt[1])
    rs3 = rs2 + _align16(rcnt[2])
    rstarts = (0, rs1, rs2, rs3)

    def rwr_body(c, offs):
        rbc = rbv[pl.ds(c * 16, 16)]
        colv = iota + c * 16
        out = []
        for b in range(NB):
            mask = rbc == b
            plsc.store_compressed(rlist.at[pl.ds(offs[b], 16)], colv, mask=mask)
            out.append(offs[b] + jnp.sum(mask.astype(jnp.int32)))
        return tuple(out)

    lax.fori_loop(0, RPW // 16, rwr_body, rstarts)

    minlc = jnp.minimum(jnp.minimum(lcnt[0], lcnt[1]),
                        jnp.minimum(lcnt[2], lcnt[3]))

    def _merge(cur, key):
        ks = jnp.sort(key)
        return jnp.sort(jnp.minimum(cur, jnp.flip(ks)))

    def _ph1_part():
        def batch_body(b, carry):
            lstart = _sel4(b, lstarts)
            lcntb = _sel4(b, lcnt)
            rstart = _sel4(b, rstarts)
            rcntb = _sel4(b, rcnt)
            nchunks = (lcntb + 15) // 16
            ngroups = (rcntb + RG - 1) // RG

            def grp_body(g, carry2):
                blk = rlist[pl.ds(rstart + (g // 4) * 16, 16)]
                ois = [_vgather(blk, _splat((g % 4) * RG + r)) & (RPW - 1)
                       for r in range(RG)]
                rxs = [plsc.load_gather(rxv, [ois[r]]) for r in range(RG)]
                rys = [plsc.load_gather(ryv, [ois[r]]) for r in range(RG)]
                rzs = [plsc.load_gather(rzv, [ois[r]]) for r in range(RG)]

                def chunk_body(c, curs):
                    out = list(curs)
                    for u in range(2):
                        o = lstart + (c * 2 + u) * 16
                        cols = lcol[pl.ds(o, 16)] & (NL - 1)
                        lxc = plsc.load_gather(lxv, [cols])
                        lyc = plsc.load_gather(lyv, [cols])
                        lzc = plsc.load_gather(lzv, [cols])
                        bad = ((c * 2 + u) * 16 + iota) >= lcntb
                        for r in range(RG):
                            dx = lxc - rxs[r]
                            dy = lyc - rys[r]
                            dz = lzc - rzs[r]
                            d2 = dx * dx + dy * dy + dz * dz
                            key = jnp.where(bad, IMAX,
                                            jnp.left_shift(d2, SHIFT) | cols)
                            out[r] = _merge(out[r], key)
                    return tuple(out)

                curs = lax.fori_loop(0, (nchunks + 1) // 2, chunk_body,
                                     (imaxv,) * RG)
                for r in range(RG):
                    oi_s = jnp.max(ois[r])

                    @pl.when(g * RG + r < rcntb)
                    def _store(r=r, oi_s=oi_s, curs=curs):
                        bestv[pl.ds(oi_s * 16, 16)] = curs[r]
                return carry2

            lax.fori_loop(0, ngroups, grp_body, 0)
            return carry

        lax.fori_loop(0, NB, batch_body, 0)

    def _ph1_full():
        def group_body(g, carry):
            r0 = g * RG
            rxs = [plsc.load_gather(rxv, [_splat(r0 + r)]) for r in range(RG)]
            rys = [plsc.load_gather(ryv, [_splat(r0 + r)]) for r in range(RG)]
            rzs = [plsc.load_gather(rzv, [_splat(r0 + r)]) for r in range(RG)]
            rbs = [plsc.load_gather(rbv, [_splat(r0 + r)]) for r in range(RG)]

            def chunk_body(c, curs):
                o = c * 16
                lxc = lxv[pl.ds(o, 16)]
                lyc = lyv[pl.ds(o, 16)]
                lzc = lzv[pl.ds(o, 16)]
                lbc = lbv[pl.ds(o, 16)]
                colv = iota + o
                bigk = colv | BIG
                out = []
                for r in range(RG):
                    dx = lxc - rxs[r]
                    dy = lyc - rys[r]
                    dz = lzc - rzs[r]
                    d2 = dx * dx + dy * dy + dz * dz
                    key = jnp.where(lbc == rbs[r],
                                    jnp.left_shift(d2, SHIFT) | colv, bigk)
                    out.append(_merge(curs[r], key))
                return tuple(out)

            curs = lax.fori_loop(0, NCH, chunk_body, (imaxv,) * RG)
            for r in range(RG):
                bestv[pl.ds((r0 + r) * 16, 16)] = curs[r]
            return carry

        lax.fori_loop(0, RPW // RG, group_body, 0)

    lax.cond(minlc >= 16, _ph1_part, _ph1_full)

    def ph2_body(i, acco):
        iv = _splat(i)
        bv = bestv[pl.ds(i * 16, 16)]
        nn = bv & (NL - 1)
        nnbuf[pl.ds(i * 16, 16)] = nn
        bv0 = _vgather(bv, iota & 0)
        m01 = jnp.where(bv0 < BIG, 1.0, 0.0)
        plsc.store_scatter(mbuf, [iv], m01, mask=iota == 0)

        rxs = plsc.load_gather(rxv, [iv])
        rys = plsc.load_gather(ryv, [iv])
        rzs = plsc.load_gather(rzv, [iv])
        fx = plsc.load_gather(lxv, [nn])
        fy = plsc.load_gather(lyv, [nn])
        fz = plsc.load_gather(lzv, [nn])
        d0 = (fz - rzs).astype(jnp.float32)
        d1 = (fy - rys).astype(jnp.float32)
        d2_ = (fx - rxs).astype(jnp.float32)
        dmap = (d0, d1, d2_)
        po = jnp.zeros((16,), jnp.float32)
        for j in range(5):
            for c in range(3):
                ps = plsc.load_gather(p3v, [iv, _splat(j * 3 + c)])
                df = ps - dmap[c]
                ad = jnp.abs(df)
                po = po + jnp.where(ad < 1.0, 0.5 * df * df, ad - 0.5)
        return acco + jnp.where(iota < TOPK, m01 * po, 0.0)

    acco = lax.fori_loop(0, RPW, ph2_body, jnp.zeros((16,), jnp.float32))

    def zero_body(i, carry):
        accb[pl.ds(i * 16, 16)] = jnp.zeros((16,), jnp.float32)
        return carry

    lax.fori_loop(0, RPW, zero_body, 0)

    for s in range(20 // NPL):
        pltpu.sync_copy(lft_h.at[pl.ds(s * NPL * NL, NPL * NL)], plv)

        def feat_body(i, carry, s=s):
            iv = _splat(i)
            nn = nnbuf[pl.ds(i * 16, 16)]
            acc = accb[pl.ds(i * 16, 16)]
            for m in range(NPL):
                asp = plsc.load_gather(av, [iv, _splat(s * NPL + m)])
                lm = plsc.load_gather(plv, [nn + m * NL])
                acc = acc + jnp.abs(asp - lm)
            accb[pl.ds(i * 16, 16)] = acc
            return carry

        lax.fori_loop(0, RPW, feat_body, 0)

    def fin_body(i, accf):
        bv = bestv[pl.ds(i * 16, 16)]
        bv0 = _vgather(bv, iota & 0)
        m01 = jnp.where(bv0 < BIG, 1.0, 0.0)
        acc = accb[pl.ds(i * 16, 16)]
        return accf + jnp.where(iota < TOPK, m01 * acc, 0.0)

    accf = lax.fori_loop(0, RPW, fin_body, jnp.zeros((16,), jnp.float32))

    pbuf[pl.ds(0, 16)] = acco
    pbuf[pl.ds(16, 16)] = accf
    pltpu.sync_copy(pbuf, part_h.at[pl.ds(wid * 32, 32)])
    pltpu.sync_copy(mbuf, matched_h.at[pl.ds(base, RPW)])


def _make_sc_fn():
    return pl.kernel(
        _sc_body,
        out_type=[jax.ShapeDtypeStruct((NR,), jnp.float32),
                  jax.ShapeDtypeStruct((NW * 32,), jnp.float32)],
        mesh=plsc.VectorSubcoreMesh(core_axis_name="c", subcore_axis_name="s",
                                    num_cores=2, num_subcores=16),
        compiler_params=pltpu.CompilerParams(needs_layout_passes=False),
        scratch_types=[
            pltpu.VMEM((NL,), jnp.int32),
            pltpu.VMEM((NL,), jnp.int32),
            pltpu.VMEM((NL,), jnp.int32),
            pltpu.VMEM((NL,), jnp.int32),
            pltpu.VMEM((RPW,), jnp.int32),
            pltpu.VMEM((RPW,), jnp.int32),
            pltpu.VMEM((RPW,), jnp.int32),
            pltpu.VMEM((RPW,), jnp.int32),
            pltpu.VMEM((RPW, 16), jnp.float32),
            pltpu.VMEM((RPW, 32), jnp.float32),
            pltpu.VMEM((RPW * 16,), jnp.int32),
            pltpu.VMEM((RPW * 16,), jnp.int32),
            pltpu.VMEM((NPL * NL,), jnp.float32),
            pltpu.VMEM((RPW * 16,), jnp.float32),
            pltpu.VMEM((RPW,), jnp.float32),
            pltpu.VMEM((32,), jnp.float32),
            pltpu.VMEM((NL + 64,), jnp.int32),
            pltpu.VMEM((RPW + 64,), jnp.int32),
            pltpu.SemaphoreType.DMA,
        ],
    )


def _tc_body(x_ref, mb_ref, mt_ref, pp_ref, o_ref):
    x = x_ref[...]
    s1 = jnp.sum(jnp.maximum(x, 0.0) + jnp.log1p(jnp.exp(-jnp.abs(x))))
    sm = jnp.sum(mb_ref[...] * x)
    m_cnt = jnp.sum(mt_ref[...])
    pp = pp_ref[...]
    col = lax.broadcasted_iota(jnp.int32, pp.shape, 1)
    offm = (col & 31) < 16
    so = jnp.sum(jnp.where(offm, pp, 0.0))
    sf = jnp.sum(jnp.where(offm, 0.0, pp))
    occ = (10.0 * s1 - 10.0 * sm) / 204800.0
    off = so / jnp.maximum(150.0 * m_cnt, 1.0)
    fe = sf / jnp.maximum(200.0 * m_cnt, 1.0)
    o_ref[...] = (0.2 * occ + off + fe).reshape(1, 1)


def kernel(logits, attrs, radar_features, lidar_features, origin, vsize_xyz,
           radar_indices, lidar_indices):
    rb = radar_indices[:, 0]
    rx = radar_indices[:, 1]
    ry = radar_indices[:, 2]
    rz = radar_indices[:, 3]
    lb = lidar_indices[:, 0]
    lx = lidar_indices[:, 1]
    ly = lidar_indices[:, 2]
    lz = lidar_indices[:, 3]

    p3 = jnp.pad(attrs[:, :, :3].reshape(NR, 15), ((0, 0), (0, 1)))
    vc = origin[None, :] + (jnp.flip(radar_indices[:, 1:4], axis=1)
                            .astype(jnp.float32) + 0.5) * vsize_xyz[None, :]
    an3 = vc[:, None, :] + attrs[:, :, :3] * vsize_xyz[None, None, :]
    a_new = jnp.concatenate([an3, attrs[:, :, 3:4]], axis=-1).reshape(NR, 20)
    a_new = jnp.pad(a_new, ((0, 0), (0, 12)))
    lft = lidar_features.T.reshape(-1)

    matched, parts = _make_sc_fn()(lx, ly, lz, lb, rx, ry, rz, rb,
                                   p3, a_new, lft)

    x2 = logits.reshape(160, 128)
    mb = jnp.repeat(matched[:, None], 5, axis=1).reshape(160, 128)
    mt = matched.reshape(32, 128)
    pp = parts.reshape(8, 128)
    out = pl.pallas_call(
        _tc_body,
        out_shape=jax.ShapeDtypeStruct((1, 1), jnp.float32),
    )(x2, mb, mt, pp)
    return out[0, 0]


# speedup vs baseline: 1.0283x; 1.0283x over previous
import jax
import jax.numpy as jnp
from jax import lax
from jax.experimental import pallas as pl
from jax.experimental.pallas import tpu as pltpu
from jax.experimental.pallas import tpu_sc as plsc

NR = 4096
NL = 8192
NB = 4
NW = 32
RPW = NR // NW
NCH = NL // 16
RG = 4
SHIFT = 13
BIG = 0x7FFFE000
IMAX = 0x7FFFFFFF
TOPK = 10
NPL = 4

_GDN = lax.GatherDimensionNumbers(
    offset_dims=(), collapsed_slice_dims=(0,), start_index_map=(0,))


def _vgather(x, idxv):
    return lax.gather(x, idxv[:, None], _GDN, (1,),
                      mode=lax.GatherScatterMode.PROMISE_IN_BOUNDS)


def _splat(val, dtype=jnp.int32):
    return jnp.full((16,), val, dtype)


def _align16(x):
    return (x + 15) & ~15


def _sel4(b, vals):
    return jnp.where(b == 0, vals[0],
                     jnp.where(b == 1, vals[1],
                               jnp.where(b == 2, vals[2], vals[3])))


def _sc_body(lx_h, ly_h, lz_h, lb_h, rx_h, ry_h, rz_h, rb_h, p3_h, a_h, lft_h,
             matched_h, part_h,
             lxv, lyv, lzv, lbv, rxv, ryv, rzv, rbv, p3v, av, bestv, nnbuf,
             plv, accb, mbuf, pbuf, lcol, rlist, sem):
    cid = lax.axis_index("c")
    sid = lax.axis_index("s")
    wid = sid * 2 + cid
    base = wid * RPW

    cps = [
        pltpu.async_copy(lx_h, lxv, sem),
        pltpu.async_copy(ly_h, lyv, sem),
        pltpu.async_copy(lz_h, lzv, sem),
        pltpu.async_copy(lb_h, lbv, sem),
        pltpu.async_copy(rx_h.at[pl.ds(base, RPW)], rxv, sem),
        pltpu.async_copy(ry_h.at[pl.ds(base, RPW)], ryv, sem),
        pltpu.async_copy(rz_h.at[pl.ds(base, RPW)], rzv, sem),
        pltpu.async_copy(rb_h.at[pl.ds(base, RPW)], rbv, sem),
        pltpu.async_copy(p3_h.at[pl.ds(base, RPW)], p3v, sem),
        pltpu.async_copy(a_h.at[pl.ds(base, RPW)], av, sem),
    ]
    for cp in cps:
        cp.wait()

    iota = lax.iota(jnp.int32, 16)
    imaxv = _splat(IMAX)

    def lcnt_body(c, cnts):
        lbc = lbv[pl.ds(c * 16, 16)]
        return tuple(cnts[b] + jnp.sum((lbc == b).astype(jnp.int32))
                     for b in range(NB))

    lcnt = lax.fori_loop(0, NCH, lcnt_body, (0, 0, 0, 0))
    ls1 = _align16(lcnt[0])
    ls2 = ls1 + _align16(lcnt[1])
    ls3 = ls2 + _align16(lcnt[2])
    lstarts = (0, ls1, ls2, ls3)

    def lwr_body(c, offs):
        lbc = lbv[pl.ds(c * 16, 16)]
        colv = iota + c * 16
        out = []
        for b in range(NB):
            mask = lbc == b
            plsc.store_compressed(lcol.at[pl.ds(offs[b], 16)], colv, mask=mask)
            out.append(offs[b] + jnp.sum(mask.astype(jnp.int32)))
        return tuple(out)

    lax.fori_loop(0, NCH, lwr_body, lstarts)

    def rcnt_body(c, cnts):
        rbc = rbv[pl.ds(c * 16, 16)]
        return tuple(cnts[b] + jnp.sum((rbc == b).astype(jnp.int32))
                     for b in range(NB))

    rcnt = lax.fori_loop(0, RPW // 16, rcnt_body, (0, 0, 0, 0))
    rs1 = _align16(rcnt[0])
    rs2 = rs1 + _align16(rcnt[1])
    rs3 = rs2 + _align16(rcnt[2])
    rstarts = (0, rs1, rs2, rs3)

    def rwr_body(c, offs):
        rbc = rbv[pl.ds(c * 16, 16)]
        colv = iota + c * 16
        out = []
        for b in range(NB):
            mask = rbc == b
            plsc.store_compressed(rlist.at[pl.ds(offs[b], 16)], colv, mask=mask)
            out.append(offs[b] + jnp.sum(mask.astype(jnp.int32)))
        return tuple(out)

    lax.fori_loop(0, RPW // 16, rwr_body, rstarts)

    minlc = jnp.minimum(jnp.minimum(lcnt[0], lcnt[1]),
                        jnp.minimum(lcnt[2], lcnt[3]))

    def _merge(cur, key):
        ks = jnp.sort(key)
        return jnp.sort(jnp.minimum(cur, jnp.flip(ks)))

    def _ph1_part():
        def batch_body(b, carry):
            lstart = _sel4(b, lstarts)
            lcntb = _sel4(b, lcnt)
            rstart = _sel4(b, rstarts)
            rcntb = _sel4(b, rcnt)
            nchunks = (lcntb + 15) // 16
            ngroups = (rcntb + RG - 1) // RG

            def grp_body(g, carry2):
                blk = rlist[pl.ds(rstart + (g // 4) * 16, 16)]
                ois = [_vgather(blk, _splat((g % 4) * RG + r)) & (RPW - 1)
                       for r in range(RG)]
                rxs = [plsc.load_gather(rxv, [ois[r]]) for r in range(RG)]
                rys = [plsc.load_gather(ryv, [ois[r]]) for r in range(RG)]
                rzs = [plsc.load_gather(rzv, [ois[r]]) for r in range(RG)]

                def chunk_body(c, curs):
                    out = list(curs)
                    for u in range(2):
                        o = lstart + (c * 2 + u) * 16
                        cols = lcol[pl.ds(o, 16)] & (NL - 1)
                        lxc = plsc.load_gather(lxv, [cols])
                        lyc = plsc.load_gather(lyv, [cols])
                        lzc = plsc.load_gather(lzv, [cols])
                        bad = ((c * 2 + u) * 16 + iota) >= lcntb
                        for r in range(RG):
                            dx = lxc - rxs[r]
                            dy = lyc - rys[r]
                            dz = lzc - rzs[r]
                            d2 = dx * dx + dy * dy + dz * dz
                            key = jnp.where(bad, IMAX,
                                            jnp.left_shift(d2, SHIFT) | cols)
                            out[r] = _merge(out[r], key)
                    return tuple(out)

                curs = lax.fori_loop(0, (nchunks + 1) // 2, chunk_body,
                                     (imaxv,) * RG)
                for r in range(RG):
                    oi_s = jnp.max(ois[r])

                    @pl.when(g * RG + r < rcntb)
                    def _store(r=r, oi_s=oi_s, curs=curs):
                        bestv[pl.ds(oi_s * 16, 16)] = curs[r]
                return carry2

            lax.fori_loop(0, ngroups, grp_body, 0)
            return carry

        lax.fori_loop(0, NB, batch_body, 0)

    def _ph1_full():
        def group_body(g, carry):
            r0 = g * RG
            rxs = [plsc.load_gather(rxv, [_splat(r0 + r)]) for r in range(RG)]
            rys = [plsc.load_gather(ryv, [_splat(r0 + r)]) for r in range(RG)]
            rzs = [plsc.load_gather(rzv, [_splat(r0 + r)]) for r in range(RG)]
            rbs = [plsc.load_gather(rbv, [_splat(r0 + r)]) for r in range(RG)]

            def chunk_body(c, curs):
                o = c * 16
                lxc = lxv[pl.ds(o, 16)]
                lyc = lyv[pl.ds(o, 16)]
                lzc = lzv[pl.ds(o, 16)]
                lbc = lbv[pl.ds(o, 16)]
                colv = iota + o
                bigk = colv | BIG
                out = []
                for r in range(RG):
                    dx = lxc - rxs[r]
                    dy = lyc - rys[r]
                    dz = lzc - rzs[r]
                    d2 = dx * dx + dy * dy + dz * dz
                    key = jnp.where(lbc == rbs[r],
                                    jnp.left_shift(d2, SHIFT) | colv, bigk)
                    out.append(_merge(curs[r], key))
                return tuple(out)

            curs = lax.fori_loop(0, NCH, chunk_body, (imaxv,) * RG)
            for r in range(RG):
                bestv[pl.ds((r0 + r) * 16, 16)] = curs[r]
            return carry

        lax.fori_loop(0, RPW // RG, group_body, 0)

    lax.cond(minlc >= 16, _ph1_part, _ph1_full)

    def ph2_body(i, acco):
        iv = _splat(i)
        bv = bestv[pl.ds(i * 16, 16)]
        nn = bv & (NL - 1)
        nnbuf[pl.ds(i * 16, 16)] = nn
        bv0 = _vgather(bv, iota & 0)
        m01 = jnp.where(bv0 < BIG, 1.0, 0.0)
        plsc.store_scatter(mbuf, [iv], m01, mask=iota == 0)

        rxs = plsc.load_gather(rxv, [iv])
        rys = plsc.load_gather(ryv, [iv])
        rzs = plsc.load_gather(rzv, [iv])
        fx = plsc.load_gather(lxv, [nn])
        fy = plsc.load_gather(lyv, [nn])
        fz = plsc.load_gather(lzv, [nn])
        d0 = (fz - rzs).astype(jnp.float32)
        d1 = (fy - rys).astype(jnp.float32)
        d2_ = (fx - rxs).astype(jnp.float32)
        dmap = (d0, d1, d2_)
        po = jnp.zeros((16,), jnp.float32)
        for j in range(5):
            for c in range(3):
                ps = plsc.load_gather(p3v, [iv, _splat(j * 3 + c)])
                df = ps - dmap[c]
                ad = jnp.abs(df)
                po = po + jnp.where(ad < 1.0, 0.5 * df * df, ad - 0.5)
        return acco + jnp.where(iota < TOPK, m01 * po, 0.0)

    acco = lax.fori_loop(0, RPW, ph2_body, jnp.zeros((16,), jnp.float32))

    def zero_body(i, carry):
        accb[pl.ds(i * 16, 16)] = jnp.zeros((16,), jnp.float32)
        return carry

    lax.fori_loop(0, RPW, zero_body, 0)

    for s in range(20 // NPL):
        pltpu.sync_copy(lft_h.at[pl.ds(s * NPL * NL, NPL * NL)], plv)

        def feat_body(i, carry, s=s):
            iv = _splat(i)
            nn = nnbuf[pl.ds(i * 16, 16)]
            acc = accb[pl.ds(i * 16, 16)]
            for m in range(NPL):
                asp = plsc.load_gather(av, [iv, _splat(s * NPL + m)])
                lm = plsc.load_gather(plv, [nn + m * NL])
                acc = acc + jnp.abs(asp - lm)
            accb[pl.ds(i * 16, 16)] = acc
            return carry

        lax.fori_loop(0, RPW, feat_body, 0)

    def fin_body(i, accf):
        bv = bestv[pl.ds(i * 16, 16)]
        bv0 = _vgather(bv, iota & 0)
        m01 = jnp.where(bv0 < BIG, 1.0, 0.0)
        acc = accb[pl.ds(i * 16, 16)]
        return accf + jnp.where(iota < TOPK, m01 * acc, 0.0)

    accf = lax.fori_loop(0, RPW, fin_body, jnp.zeros((16,), jnp.float32))

    pbuf[pl.ds(0, 16)] = acco
    pbuf[pl.ds(16, 16)] = accf
    pltpu.sync_copy(pbuf, part_h.at[pl.ds(wid * 32, 32)])
    pltpu.sync_copy(mbuf, matched_h.at[pl.ds(base, RPW)])


def _make_sc_fn():
    return pl.kernel(
        _sc_body,
        out_type=[jax.ShapeDtypeStruct((NR,), jnp.float32),
                  jax.ShapeDtypeStruct((NW * 32,), jnp.float32)],
        mesh=plsc.VectorSubcoreMesh(core_axis_name="c", subcore_axis_name="s",
                                    num_cores=2, num_subcores=16),
        compiler_params=pltpu.CompilerParams(needs_layout_passes=False),
        scratch_types=[
            pltpu.VMEM((NL,), jnp.int32),
            pltpu.VMEM((NL,), jnp.int32),
            pltpu.VMEM((NL,), jnp.int32),
            pltpu.VMEM((NL,), jnp.int32),
            pltpu.VMEM((RPW,), jnp.int32),
            pltpu.VMEM((RPW,), jnp.int32),
            pltpu.VMEM((RPW,), jnp.int32),
            pltpu.VMEM((RPW,), jnp.int32),
            pltpu.VMEM((RPW, 16), jnp.float32),
            pltpu.VMEM((RPW, 32), jnp.float32),
            pltpu.VMEM((RPW * 16,), jnp.int32),
            pltpu.VMEM((RPW * 16,), jnp.int32),
            pltpu.VMEM((NPL * NL,), jnp.float32),
            pltpu.VMEM((RPW * 16,), jnp.float32),
            pltpu.VMEM((RPW,), jnp.float32),
            pltpu.VMEM((32,), jnp.float32),
            pltpu.VMEM((NL + 64,), jnp.int32),
            pltpu.VMEM((RPW + 64,), jnp.int32),
            pltpu.SemaphoreType.DMA,
        ],
    )


def _tc_body(x_ref, mb_ref, mt_ref, pp_ref, o_ref):
    x = x_ref[...]
    s1 = jnp.sum(jnp.maximum(x, 0.0) + jnp.log1p(jnp.exp(-jnp.abs(x))))
    sm = jnp.sum(mb_ref[...] * x)
    m_cnt = jnp.sum(mt_ref[...])
    pp = pp_ref[...]
    col = lax.broadcasted_iota(jnp.int32, pp.shape, 1)
    offm = (col & 31) < 16
    so = jnp.sum(jnp.where(offm, pp, 0.0))
    sf = jnp.sum(jnp.where(offm, 0.0, pp))
    occ = (10.0 * s1 - 10.0 * sm) / 204800.0
    off = so / jnp.maximum(150.0 * m_cnt, 1.0)
    fe = sf / jnp.maximum(200.0 * m_cnt, 1.0)
    o_ref[...] = (0.2 * occ + off + fe).reshape(1, 1)


def kernel(logits, attrs, radar_features, lidar_features, origin, vsize_xyz,
           radar_indices, lidar_indices):
    rb = radar_indices[:, 0]
    rx = radar_indices[:, 1]
    ry = radar_indices[:, 2]
    rz = radar_indices[:, 3]
    lb = lidar_indices[:, 0]
    lx = lidar_indices[:, 1]
    ly = lidar_indices[:, 2]
    lz = lidar_indices[:, 3]

    p3 = jnp.pad(attrs[:, :, :3].reshape(NR, 15), ((0, 0), (0, 1)))
    vc = origin[None, :] + (jnp.flip(radar_indices[:, 1:4], axis=1)
                            .astype(jnp.float32) + 0.5) * vsize_xyz[None, :]
    an3 = vc[:, None, :] + attrs[:, :, :3] * vsize_xyz[None, None, :]
    a_new = jnp.concatenate([an3, attrs[:, :, 3:4]], axis=-1).reshape(NR, 20)
    a_new = jnp.pad(a_new, ((0, 0), (0, 12)))
    lft = lidar_features.T.reshape(-1)

    matched, parts = _make_sc_fn()(lx, ly, lz, lb, rx, ry, rz, rb,
                                   p3, a_new, lft)

    x2 = logits.reshape(160, 128)
    mb = jnp.repeat(matched[:, None], 5, axis=1).reshape(160, 128)
    mt = matched.reshape(32, 128)
    pp = parts.reshape(8, 128)
    out = pl.pallas_call(
        _tc_body,
        out_shape=jax.ShapeDtypeStruct((1, 1), jnp.float32),
    )(x2, mb, mt, pp)
    return out[0, 0]


# speedup vs baseline: 1.0546x; 1.0256x over previous
import jax
import jax.numpy as jnp
from jax import lax
from jax.experimental import pallas as pl
from jax.experimental.pallas import tpu as pltpu
from jax.experimental.pallas import tpu_sc as plsc

NR = 4096
NL = 8192
NB = 4
NW = 32
RPW = NR // NW
NCH = NL // 16
RG = 4
SHIFT = 13
BIG = 0x7FFFE000
IMAX = 0x7FFFFFFF
TOPK = 10
NPL = 2

_GDN = lax.GatherDimensionNumbers(
    offset_dims=(), collapsed_slice_dims=(0,), start_index_map=(0,))


def _vgather(x, idxv):
    return lax.gather(x, idxv[:, None], _GDN, (1,),
                      mode=lax.GatherScatterMode.PROMISE_IN_BOUNDS)


def _splat(val, dtype=jnp.int32):
    return jnp.full((16,), val, dtype)


def _align16(x):
    return (x + 15) & ~15


def _sel4(b, vals):
    return jnp.where(b == 0, vals[0],
                     jnp.where(b == 1, vals[1],
                               jnp.where(b == 2, vals[2], vals[3])))


def _sc_body(lt_h, rt_h, p3_h, a_h, lft_h,
             matched_h, part_h,
             lxv, lyv, lzv, lbv, rxv, ryv, rzv, rbv, p3v, av, bestv, nnbuf,
             plva, plvb, accb, mbuf, pbuf, lcol, rlist, sem, sem2):
    cid = lax.axis_index("c")
    sid = lax.axis_index("s")
    wid = sid * 2 + cid
    base = wid * RPW

    plcp = pltpu.async_copy(lft_h.at[pl.ds(0, NPL * NL)], plva, sem2)
    cps = [
        pltpu.async_copy(lt_h.at[pl.ds(0, NL)], lbv, sem),
        pltpu.async_copy(lt_h.at[pl.ds(NL, NL)], lxv, sem),
        pltpu.async_copy(lt_h.at[pl.ds(2 * NL, NL)], lyv, sem),
        pltpu.async_copy(lt_h.at[pl.ds(3 * NL, NL)], lzv, sem),
        pltpu.async_copy(rt_h.at[pl.ds(base, RPW)], rbv, sem),
        pltpu.async_copy(rt_h.at[pl.ds(NR + base, RPW)], rxv, sem),
        pltpu.async_copy(rt_h.at[pl.ds(2 * NR + base, RPW)], ryv, sem),
        pltpu.async_copy(rt_h.at[pl.ds(3 * NR + base, RPW)], rzv, sem),
        pltpu.async_copy(p3_h.at[pl.ds(base, RPW)], p3v, sem),
        pltpu.async_copy(a_h.at[pl.ds(base, RPW)], av, sem),
    ]
    for cp in cps:
        cp.wait()

    iota = lax.iota(jnp.int32, 16)
    imaxv = _splat(IMAX)

    def lcnt_body(c, cnts):
        lbc = lbv[pl.ds(c * 16, 16)]
        return tuple(cnts[b] + jnp.sum((lbc == b).astype(jnp.int32))
                     for b in range(NB))

    lcnt = lax.fori_loop(0, NCH, lcnt_body, (0, 0, 0, 0))
    ls1 = _align16(lcnt[0])
    ls2 = ls1 + _align16(lcnt[1])
    ls3 = ls2 + _align16(lcnt[2])
    lstarts = (0, ls1, ls2, ls3)

    def lwr_body(c, offs):
        lbc = lbv[pl.ds(c * 16, 16)]
        colv = iota + c * 16
        out = []
        for b in range(NB):
            mask = lbc == b
            plsc.store_compressed(lcol.at[pl.ds(offs[b], 16)], colv, mask=mask)
            out.append(offs[b] + jnp.sum(mask.astype(jnp.int32)))
        return tuple(out)

    lax.fori_loop(0, NCH, lwr_body, lstarts)

    def rcnt_body(c, cnts):
        rbc = rbv[pl.ds(c * 16, 16)]
        return tuple(cnts[b] + jnp.sum((rbc == b).astype(jnp.int32))
                     for b in range(NB))

    rcnt = lax.fori_loop(0, RPW // 16, rcnt_body, (0, 0, 0, 0))
    rs1 = _align16(rcnt[0])
    rs2 = rs1 + _align16(rcnt[1])
    rs3 = rs2 + _align16(rcnt[2])
    rstarts = (0, rs1, rs2, rs3)

    def rwr_body(c, offs):
        rbc = rbv[pl.ds(c * 16, 16)]
        colv = iota + c * 16
        out = []
        for b in range(NB):
            mask = rbc == b
            plsc.store_compressed(rlist.at[pl.ds(offs[b], 16)], colv, mask=mask)
            out.append(offs[b] + jnp.sum(mask.astype(jnp.int32)))
        return tuple(out)

    lax.fori_loop(0, RPW // 16, rwr_body, rstarts)

    minlc = jnp.minimum(jnp.minimum(lcnt[0], lcnt[1]),
                        jnp.minimum(lcnt[2], lcnt[3]))

    def _merge(cur, key):
        ks = jnp.sort(key)
        return jnp.sort(jnp.minimum(cur, jnp.flip(ks)))

    def _ph1_part():
        def batch_body(b, carry):
            lstart = _sel4(b, lstarts)
            lcntb = _sel4(b, lcnt)
            rstart = _sel4(b, rstarts)
            rcntb = _sel4(b, rcnt)
            nchunks = (lcntb + 15) // 16
            ngroups = (rcntb + RG - 1) // RG

            def grp_body(g, carry2):
                blk = rlist[pl.ds(rstart + (g // 4) * 16, 16)]
                ois = [_vgather(blk, _splat((g % 4) * RG + r)) & (RPW - 1)
                       for r in range(RG)]
                rxs = [plsc.load_gather(rxv, [ois[r]]) for r in range(RG)]
                rys = [plsc.load_gather(ryv, [ois[r]]) for r in range(RG)]
                rzs = [plsc.load_gather(rzv, [ois[r]]) for r in range(RG)]

                def chunk_body(c, curs):
                    out = list(curs)
                    for u in range(2):
                        o = lstart + (c * 2 + u) * 16
                        cols = lcol[pl.ds(o, 16)] & (NL - 1)
                        lxc = plsc.load_gather(lxv, [cols])
                        lyc = plsc.load_gather(lyv, [cols])
                        lzc = plsc.load_gather(lzv, [cols])
                        bad = ((c * 2 + u) * 16 + iota) >= lcntb
                        for r in range(RG):
                            dx = lxc - rxs[r]
                            dy = lyc - rys[r]
                            dz = lzc - rzs[r]
                            d2 = dx * dx + dy * dy + dz * dz
                            key = jnp.where(bad, IMAX,
                                            jnp.left_shift(d2, SHIFT) | cols)
                            out[r] = _merge(out[r], key)
                    return tuple(out)

                curs = lax.fori_loop(0, (nchunks + 1) // 2, chunk_body,
                                     (imaxv,) * RG)
                for r in range(RG):
                    oi_s = jnp.max(ois[r])

                    @pl.when(g * RG + r < rcntb)
                    def _store(r=r, oi_s=oi_s, curs=curs):
                        bestv[pl.ds(oi_s * 16, 16)] = curs[r]
                return carry2

            lax.fori_loop(0, ngroups, grp_body, 0)
            return carry

        lax.fori_loop(0, NB, batch_body, 0)

    def _ph1_full():
        def group_body(g, carry):
            r0 = g * RG
            rxs = [plsc.load_gather(rxv, [_splat(r0 + r)]) for r in range(RG)]
            rys = [plsc.load_gather(ryv, [_splat(r0 + r)]) for r in range(RG)]
            rzs = [plsc.load_gather(rzv, [_splat(r0 + r)]) for r in range(RG)]
            rbs = [plsc.load_gather(rbv, [_splat(r0 + r)]) for r in range(RG)]

            def chunk_body(c, curs):
                o = c * 16
                lxc = lxv[pl.ds(o, 16)]
                lyc = lyv[pl.ds(o, 16)]
                lzc = lzv[pl.ds(o, 16)]
                lbc = lbv[pl.ds(o, 16)]
                colv = iota + o
                bigk = colv | BIG
                out = []
                for r in range(RG):
                    dx = lxc - rxs[r]
                    dy = lyc - rys[r]
                    dz = lzc - rzs[r]
                    d2 = dx * dx + dy * dy + dz * dz
                    key = jnp.where(lbc == rbs[r],
                                    jnp.left_shift(d2, SHIFT) | colv, bigk)
                    out.append(_merge(curs[r], key))
                return tuple(out)

            curs = lax.fori_loop(0, NCH, chunk_body, (imaxv,) * RG)
            for r in range(RG):
                bestv[pl.ds((r0 + r) * 16, 16)] = curs[r]
            return carry

        lax.fori_loop(0, RPW // RG, group_body, 0)

    lax.cond(minlc >= 16, _ph1_part, _ph1_full)

    def ph2_body(i, acco):
        iv = _splat(i)
        bv = bestv[pl.ds(i * 16, 16)]
        nn = bv & (NL - 1)
        nnbuf[pl.ds(i * 16, 16)] = nn
        bv0 = _vgather(bv, iota & 0)
        m01 = jnp.where(bv0 < BIG, 1.0, 0.0)
        plsc.store_scatter(mbuf, [iv], m01, mask=iota == 0)

        rxs = plsc.load_gather(rxv, [iv])
        rys = plsc.load_gather(ryv, [iv])
        rzs = plsc.load_gather(rzv, [iv])
        fx = plsc.load_gather(lxv, [nn])
        fy = plsc.load_gather(lyv, [nn])
        fz = plsc.load_gather(lzv, [nn])
        d0 = (fz - rzs).astype(jnp.float32)
        d1 = (fy - rys).astype(jnp.float32)
        d2_ = (fx - rxs).astype(jnp.float32)
        dmap = (d0, d1, d2_)
        po = jnp.zeros((16,), jnp.float32)
        for j in range(5):
            for c in range(3):
                ps = plsc.load_gather(p3v, [iv, _splat(j * 3 + c)])
                df = ps - dmap[c]
                ad = jnp.abs(df)
                po = po + jnp.where(ad < 1.0, 0.5 * df * df, ad - 0.5)
        return acco + jnp.where(iota < TOPK, m01 * po, 0.0)

    acco = lax.fori_loop(0, RPW, ph2_body, jnp.zeros((16,), jnp.float32))

    def zero_body(i, carry):
        accb[pl.ds(i * 16, 16)] = jnp.zeros((16,), jnp.float32)
        return carry

    lax.fori_loop(0, RPW, zero_body, 0)

    nstages = 20 // NPL
    bufs = (plva, plvb)
    for s in range(nstages):
        plcp.wait()
        if s + 1 < nstages:
            plcp = pltpu.async_copy(
                lft_h.at[pl.ds((s + 1) * NPL * NL, NPL * NL)],
                bufs[(s + 1) % 2], sem2)
        plv = bufs[s % 2]

        def feat_body(i, carry, s=s, plv=plv):
            iv = _splat(i)
            nn = nnbuf[pl.ds(i * 16, 16)]
            acc = accb[pl.ds(i * 16, 16)]
            for m in range(NPL):
                asp = plsc.load_gather(av, [iv, _splat(s * NPL + m)])
                lm = plsc.load_gather(plv, [nn + m * NL])
                acc = acc + jnp.abs(asp - lm)
            accb[pl.ds(i * 16, 16)] = acc
            return carry

        lax.fori_loop(0, RPW, feat_body, 0)

    def fin_body(i, accf):
        bv = bestv[pl.ds(i * 16, 16)]
        bv0 = _vgather(bv, iota & 0)
        m01 = jnp.where(bv0 < BIG, 1.0, 0.0)
        acc = accb[pl.ds(i * 16, 16)]
        return accf + jnp.where(iota < TOPK, m01 * acc, 0.0)

    accf = lax.fori_loop(0, RPW, fin_body, jnp.zeros((16,), jnp.float32))

    pbuf[pl.ds(0, 16)] = acco
    pbuf[pl.ds(16, 16)] = accf
    pltpu.sync_copy(pbuf, part_h.at[pl.ds(wid * 32, 32)])
    pltpu.sync_copy(mbuf, matched_h.at[pl.ds(base, RPW)])


def _make_sc_fn():
    return pl.kernel(
        _sc_body,
        out_type=[jax.ShapeDtypeStruct((NR,), jnp.float32),
                  jax.ShapeDtypeStruct((NW * 32,), jnp.float32)],
        mesh=plsc.VectorSubcoreMesh(core_axis_name="c", subcore_axis_name="s",
                                    num_cores=2, num_subcores=16),
        compiler_params=pltpu.CompilerParams(needs_layout_passes=False),
        scratch_types=[
            pltpu.VMEM((NL,), jnp.int32),
            pltpu.VMEM((NL,), jnp.int32),
            pltpu.VMEM((NL,), jnp.int32),
            pltpu.VMEM((NL,), jnp.int32),
            pltpu.VMEM((RPW,), jnp.int32),
            pltpu.VMEM((RPW,), jnp.int32),
            pltpu.VMEM((RPW,), jnp.int32),
            pltpu.VMEM((RPW,), jnp.int32),
            pltpu.VMEM((RPW, 16), jnp.float32),
            pltpu.VMEM((RPW, 32), jnp.float32),
            pltpu.VMEM((RPW * 16,), jnp.int32),
            pltpu.VMEM((RPW * 16,), jnp.int32),
            pltpu.VMEM((NPL * NL,), jnp.float32),
            pltpu.VMEM((NPL * NL,), jnp.float32),
            pltpu.VMEM((RPW * 16,), jnp.float32),
            pltpu.VMEM((RPW,), jnp.float32),
            pltpu.VMEM((32,), jnp.float32),
            pltpu.VMEM((NL + 64,), jnp.int32),
            pltpu.VMEM((RPW + 64,), jnp.int32),
            pltpu.SemaphoreType.DMA,
            pltpu.SemaphoreType.DMA,
        ],
    )


def _tc_body(x_ref, mb_ref, mt_ref, pp_ref, o_ref):
    x = x_ref[...]
    s1 = jnp.sum(jnp.maximum(x, 0.0) + jnp.log1p(jnp.exp(-jnp.abs(x))))
    sm = jnp.sum(mb_ref[...] * x)
    m_cnt = jnp.sum(mt_ref[...])
    pp = pp_ref[...]
    col = lax.broadcasted_iota(jnp.int32, pp.shape, 1)
    offm = (col & 31) < 16
    so = jnp.sum(jnp.where(offm, pp, 0.0))
    sf = jnp.sum(jnp.where(offm, 0.0, pp))
    occ = (10.0 * s1 - 10.0 * sm) / 204800.0
    off = so / jnp.maximum(150.0 * m_cnt, 1.0)
    fe = sf / jnp.maximum(200.0 * m_cnt, 1.0)
    o_ref[...] = (0.2 * occ + off + fe).reshape(1, 1)


def kernel(logits, attrs, radar_features, lidar_features, origin, vsize_xyz,
           radar_indices, lidar_indices):
    rt = radar_indices.T.reshape(-1)
    lt = lidar_indices.T.reshape(-1)

    p3 = jnp.pad(attrs[:, :, :3].reshape(NR, 15), ((0, 0), (0, 1)))
    vc = origin[None, :] + (jnp.flip(radar_indices[:, 1:4], axis=1)
                            .astype(jnp.float32) + 0.5) * vsize_xyz[None, :]
    an3 = vc[:, None, :] + attrs[:, :, :3] * vsize_xyz[None, None, :]
    a_new = jnp.concatenate([an3, attrs[:, :, 3:4]], axis=-1).reshape(NR, 20)
    a_new = jnp.pad(a_new, ((0, 0), (0, 12)))
    lft = lidar_features.T.reshape(-1)

    matched, parts = _make_sc_fn()(lt, rt, p3, a_new, lft)

    x2 = logits.reshape(160, 128)
    mb = jnp.repeat(matched[:, None], 5, axis=1).reshape(160, 128)
    mt = matched.reshape(32, 128)
    pp = parts.reshape(8, 128)
    out = pl.pallas_call(
        _tc_body,
        out_shape=jax.ShapeDtypeStruct((1, 1), jnp.float32),
    )(x2, mb, mt, pp)
    return out[0, 0]


# speedup vs baseline: 1.0835x; 1.0274x over previous
import jax
import jax.numpy as jnp
from jax import lax
from jax.experimental import pallas as pl
from jax.experimental.pallas import tpu as pltpu
from jax.experimental.pallas import tpu_sc as plsc

NR = 4096
NL = 8192
NB = 4
NW = 32
RPW = NR // NW
NCH = NL // 16
RG = 4
SHIFT = 13
BIG = 0x7FFFE000
IMAX = 0x7FFFFFFF
TOPK = 10
NPL = 2

_GDN = lax.GatherDimensionNumbers(
    offset_dims=(), collapsed_slice_dims=(0,), start_index_map=(0,))


def _vgather(x, idxv):
    return lax.gather(x, idxv[:, None], _GDN, (1,),
                      mode=lax.GatherScatterMode.PROMISE_IN_BOUNDS)


def _splat(val, dtype=jnp.int32):
    return jnp.full((16,), val, dtype)


def _align16(x):
    return (x + 15) & ~15


def _sel4(b, vals):
    return jnp.where(b == 0, vals[0],
                     jnp.where(b == 1, vals[1],
                               jnp.where(b == 2, vals[2], vals[3])))


def _sc_body(lt_h, rt_h, p3_h, a_h, lft_h,
             matched_h, part_h,
             lxv, lyv, lzv, lbv, rxv, ryv, rzv, rbv, p3v, av, bestv, nnbuf,
             plva, plvb, accb, mbuf, pbuf, lcol, rlist, sem, sem2):
    cid = lax.axis_index("c")
    sid = lax.axis_index("s")
    wid = sid * 2 + cid
    base = wid * RPW

    plcp = pltpu.async_copy(lft_h.at[pl.ds(0, NPL * NL)], plva, sem2)
    cps = [
        pltpu.async_copy(lt_h.at[pl.ds(0, NL)], lbv, sem),
        pltpu.async_copy(lt_h.at[pl.ds(NL, NL)], lxv, sem),
        pltpu.async_copy(lt_h.at[pl.ds(2 * NL, NL)], lyv, sem),
        pltpu.async_copy(lt_h.at[pl.ds(3 * NL, NL)], lzv, sem),
        pltpu.async_copy(rt_h.at[pl.ds(base, RPW)], rbv, sem),
        pltpu.async_copy(rt_h.at[pl.ds(NR + base, RPW)], rxv, sem),
        pltpu.async_copy(rt_h.at[pl.ds(2 * NR + base, RPW)], ryv, sem),
        pltpu.async_copy(rt_h.at[pl.ds(3 * NR + base, RPW)], rzv, sem),
        pltpu.async_copy(p3_h.at[pl.ds(base, RPW)], p3v, sem),
        pltpu.async_copy(a_h.at[pl.ds(base, RPW)], av, sem),
    ]
    for cp in cps:
        cp.wait()

    iota = lax.iota(jnp.int32, 16)
    imaxv = _splat(IMAX)

    def lcnt_body(c, cnts):
        lbc = lbv[pl.ds(c * 16, 16)]
        return tuple(cnts[b] + jnp.sum((lbc == b).astype(jnp.int32))
                     for b in range(NB))

    lcnt = lax.fori_loop(0, NCH, lcnt_body, (0, 0, 0, 0))
    ls1 = _align16(lcnt[0])
    ls2 = ls1 + _align16(lcnt[1])
    ls3 = ls2 + _align16(lcnt[2])
    lstarts = (0, ls1, ls2, ls3)

    def lwr_body(c, offs):
        lbc = lbv[pl.ds(c * 16, 16)]
        colv = iota + c * 16
        out = []
        for b in range(NB):
            mask = lbc == b
            plsc.store_compressed(lcol.at[pl.ds(offs[b], 16)], colv, mask=mask)
            out.append(offs[b] + jnp.sum(mask.astype(jnp.int32)))
        return tuple(out)

    lax.fori_loop(0, NCH, lwr_body, lstarts)

    def rcnt_body(c, cnts):
        rbc = rbv[pl.ds(c * 16, 16)]
        return tuple(cnts[b] + jnp.sum((rbc == b).astype(jnp.int32))
                     for b in range(NB))

    rcnt = lax.fori_loop(0, RPW // 16, rcnt_body, (0, 0, 0, 0))
    rs1 = _align16(rcnt[0])
    rs2 = rs1 + _align16(rcnt[1])
    rs3 = rs2 + _align16(rcnt[2])
    rstarts = (0, rs1, rs2, rs3)

    def rwr_body(c, offs):
        rbc = rbv[pl.ds(c * 16, 16)]
        colv = iota + c * 16
        out = []
        for b in range(NB):
            mask = rbc == b
            plsc.store_compressed(rlist.at[pl.ds(offs[b], 16)], colv, mask=mask)
            out.append(offs[b] + jnp.sum(mask.astype(jnp.int32)))
        return tuple(out)

    lax.fori_loop(0, RPW // 16, rwr_body, rstarts)

    minlc = jnp.minimum(jnp.minimum(lcnt[0], lcnt[1]),
                        jnp.minimum(lcnt[2], lcnt[3]))

    def _merge(cur, key):
        ks = jnp.sort(key)
        return jnp.sort(jnp.minimum(cur, jnp.flip(ks)))

    def _ph1_part():
        def batch_body(b, carry):
            lstart = _sel4(b, lstarts)
            lcntb = _sel4(b, lcnt)
            rstart = _sel4(b, rstarts)
            rcntb = _sel4(b, rcnt)
            nchunks = (lcntb + 15) // 16
            ngroups = (rcntb + RG - 1) // RG

            def grp_body(g, carry2):
                blk = rlist[pl.ds(rstart + (g // 4) * 16, 16)]
                ois = [_vgather(blk, _splat((g % 4) * RG + r)) & (RPW - 1)
                       for r in range(RG)]
                rxs = [plsc.load_gather(rxv, [ois[r]]) for r in range(RG)]
                rys = [plsc.load_gather(ryv, [ois[r]]) for r in range(RG)]
                rzs = [plsc.load_gather(rzv, [ois[r]]) for r in range(RG)]

                def chunk_body(c, curs):
                    out = list(curs)
                    for u in range(2):
                        o = lstart + (c * 2 + u) * 16
                        cols = lcol[pl.ds(o, 16)] & (NL - 1)
                        lxc = plsc.load_gather(lxv, [cols])
                        lyc = plsc.load_gather(lyv, [cols])
                        lzc = plsc.load_gather(lzv, [cols])
                        bad = ((c * 2 + u) * 16 + iota) >= lcntb
                        for r in range(RG):
                            dx = lxc - rxs[r]
                            dy = lyc - rys[r]
                            dz = lzc - rzs[r]
                            d2 = dx * dx + dy * dy + dz * dz
                            key = jnp.where(bad, IMAX,
                                            jnp.left_shift(d2, SHIFT) | cols)
                            out[r] = _merge(out[r], key)
                    return tuple(out)

                curs = lax.fori_loop(0, (nchunks + 1) // 2, chunk_body,
                                     (imaxv,) * RG)
                for r in range(RG):
                    oi_s = jnp.max(ois[r])

                    @pl.when(g * RG + r < rcntb)
                    def _store(r=r, oi_s=oi_s, curs=curs):
                        bestv[pl.ds(oi_s * 16, 16)] = curs[r]
                return carry2

            lax.fori_loop(0, ngroups, grp_body, 0)
            return carry

        lax.fori_loop(0, NB, batch_body, 0)

    def _ph1_full():
        def group_body(g, carry):
            r0 = g * RG
            rxs = [plsc.load_gather(rxv, [_splat(r0 + r)]) for r in range(RG)]
            rys = [plsc.load_gather(ryv, [_splat(r0 + r)]) for r in range(RG)]
            rzs = [plsc.load_gather(rzv, [_splat(r0 + r)]) for r in range(RG)]
            rbs = [plsc.load_gather(rbv, [_splat(r0 + r)]) for r in range(RG)]

            def chunk_body(c, curs):
                o = c * 16
                lxc = lxv[pl.ds(o, 16)]
                lyc = lyv[pl.ds(o, 16)]
                lzc = lzv[pl.ds(o, 16)]
                lbc = lbv[pl.ds(o, 16)]
                colv = iota + o
                bigk = colv | BIG
                out = []
                for r in range(RG):
                    dx = lxc - rxs[r]
                    dy = lyc - rys[r]
                    dz = lzc - rzs[r]
                    d2 = dx * dx + dy * dy + dz * dz
                    key = jnp.where(lbc == rbs[r],
                                    jnp.left_shift(d2, SHIFT) | colv, bigk)
                    out.append(_merge(curs[r], key))
                return tuple(out)

            curs = lax.fori_loop(0, NCH, chunk_body, (imaxv,) * RG)
            for r in range(RG):
                bestv[pl.ds((r0 + r) * 16, 16)] = curs[r]
            return carry

        lax.fori_loop(0, RPW // RG, group_body, 0)

    lax.cond(minlc >= 16, _ph1_part, _ph1_full)

    def ph2_body(i, acco):
        iv = _splat(i)
        bv = bestv[pl.ds(i * 16, 16)]
        nn = bv & (NL - 1)
        nnbuf[pl.ds(i * 16, 16)] = nn
        bv0 = _vgather(bv, iota & 0)
        m01 = jnp.where(bv0 < BIG, 1.0, 0.0)
        plsc.store_scatter(mbuf, [iv], m01, mask=iota == 0)

        rxs = plsc.load_gather(rxv, [iv])
        rys = plsc.load_gather(ryv, [iv])
        rzs = plsc.load_gather(rzv, [iv])
        fx = plsc.load_gather(lxv, [nn])
        fy = plsc.load_gather(lyv, [nn])
        fz = plsc.load_gather(lzv, [nn])
        d0 = (fz - rzs).astype(jnp.float32)
        d1 = (fy - rys).astype(jnp.float32)
        d2_ = (fx - rxs).astype(jnp.float32)
        dmap = (d0, d1, d2_)
        prow = p3v[i]
        po = jnp.zeros((16,), jnp.float32)
        for j in range(5):
            for c in range(3):
                ps = _vgather(prow, _splat(j * 3 + c))
                df = ps - dmap[c]
                ad = jnp.abs(df)
                po = po + jnp.where(ad < 1.0, 0.5 * df * df, ad - 0.5)
        return acco + jnp.where(iota < TOPK, m01 * po, 0.0)

    acco = lax.fori_loop(0, RPW, ph2_body, jnp.zeros((16,), jnp.float32))

    nstages = 20 // NPL
    bufs = (plva, plvb)
    accf = jnp.zeros((16,), jnp.float32)
    for s in range(nstages):
        plcp.wait()
        if s + 1 < nstages:
            plcp = pltpu.async_copy(
                lft_h.at[pl.ds((s + 1) * NPL * NL, NPL * NL)],
                bufs[(s + 1) % 2], sem2)
        plv = bufs[s % 2]
        first, last = s == 0, s == nstages - 1

        def feat_body(i, accf, s=s, plv=plv, first=first, last=last):
            iv = _splat(i)
            nn = nnbuf[pl.ds(i * 16, 16)]
            if first:
                acc = jnp.zeros((16,), jnp.float32)
            else:
                acc = accb[pl.ds(i * 16, 16)]
            for m in range(NPL):
                asp = plsc.load_gather(av, [iv, _splat(s * NPL + m)])
                lm = plsc.load_gather(plv, [nn + m * NL])
                acc = acc + jnp.abs(asp - lm)
            if last:
                bv = bestv[pl.ds(i * 16, 16)]
                bv0 = _vgather(bv, iota & 0)
                m01 = jnp.where(bv0 < BIG, 1.0, 0.0)
                return accf + jnp.where(iota < TOPK, m01 * acc, 0.0)
            accb[pl.ds(i * 16, 16)] = acc
            return accf

        accf = lax.fori_loop(0, RPW, feat_body, accf)

    pbuf[pl.ds(0, 16)] = acco
    pbuf[pl.ds(16, 16)] = accf
    pltpu.sync_copy(pbuf, part_h.at[pl.ds(wid * 32, 32)])
    pltpu.sync_copy(mbuf, matched_h.at[pl.ds(base, RPW)])


def _make_sc_fn():
    return pl.kernel(
        _sc_body,
        out_type=[jax.ShapeDtypeStruct((NR,), jnp.float32),
                  jax.ShapeDtypeStruct((NW * 32,), jnp.float32)],
        mesh=plsc.VectorSubcoreMesh(core_axis_name="c", subcore_axis_name="s",
                                    num_cores=2, num_subcores=16),
        compiler_params=pltpu.CompilerParams(needs_layout_passes=False),
        scratch_types=[
            pltpu.VMEM((NL,), jnp.int32),
            pltpu.VMEM((NL,), jnp.int32),
            pltpu.VMEM((NL,), jnp.int32),
            pltpu.VMEM((NL,), jnp.int32),
            pltpu.VMEM((RPW,), jnp.int32),
            pltpu.VMEM((RPW,), jnp.int32),
            pltpu.VMEM((RPW,), jnp.int32),
            pltpu.VMEM((RPW,), jnp.int32),
            pltpu.VMEM((RPW, 16), jnp.float32),
            pltpu.VMEM((RPW, 32), jnp.float32),
            pltpu.VMEM((RPW * 16,), jnp.int32),
            pltpu.VMEM((RPW * 16,), jnp.int32),
            pltpu.VMEM((NPL * NL,), jnp.float32),
            pltpu.VMEM((NPL * NL,), jnp.float32),
            pltpu.VMEM((RPW * 16,), jnp.float32),
            pltpu.VMEM((RPW,), jnp.float32),
            pltpu.VMEM((32,), jnp.float32),
            pltpu.VMEM((NL + 64,), jnp.int32),
            pltpu.VMEM((RPW + 64,), jnp.int32),
            pltpu.SemaphoreType.DMA,
            pltpu.SemaphoreType.DMA,
        ],
    )


def _tc_body(x_ref, mb_ref, mt_ref, pp_ref, o_ref):
    x = x_ref[...]
    s1 = jnp.sum(jnp.maximum(x, 0.0) + jnp.log1p(jnp.exp(-jnp.abs(x))))
    sm = jnp.sum(mb_ref[...] * x)
    m_cnt = jnp.sum(mt_ref[...])
    pp = pp_ref[...]
    col = lax.broadcasted_iota(jnp.int32, pp.shape, 1)
    offm = (col & 31) < 16
    so = jnp.sum(jnp.where(offm, pp, 0.0))
    sf = jnp.sum(jnp.where(offm, 0.0, pp))
    occ = (10.0 * s1 - 10.0 * sm) / 204800.0
    off = so / jnp.maximum(150.0 * m_cnt, 1.0)
    fe = sf / jnp.maximum(200.0 * m_cnt, 1.0)
    o_ref[...] = (0.2 * occ + off + fe).reshape(1, 1)


def kernel(logits, attrs, radar_features, lidar_features, origin, vsize_xyz,
           radar_indices, lidar_indices):
    rt = radar_indices.T.reshape(-1)
    lt = lidar_indices.T.reshape(-1)

    p3 = jnp.pad(attrs[:, :, :3].reshape(NR, 15), ((0, 0), (0, 1)))
    vc = origin[None, :] + (jnp.flip(radar_indices[:, 1:4], axis=1)
                            .astype(jnp.float32) + 0.5) * vsize_xyz[None, :]
    an3 = vc[:, None, :] + attrs[:, :, :3] * vsize_xyz[None, None, :]
    a_new = jnp.concatenate([an3, attrs[:, :, 3:4]], axis=-1).reshape(NR, 20)
    a_new = jnp.pad(a_new, ((0, 0), (0, 12)))
    lft = lidar_features.T.reshape(-1)

    matched, parts = _make_sc_fn()(lt, rt, p3, a_new, lft)

    x2 = logits.reshape(160, 128)
    mb = jnp.repeat(matched[:, None], 5, axis=1).reshape(160, 128)
    mt = matched.reshape(32, 128)
    pp = parts.reshape(8, 128)
    out = pl.pallas_call(
        _tc_body,
        out_shape=jax.ShapeDtypeStruct((1, 1), jnp.float32),
    )(x2, mb, mt, pp)
    return out[0, 0]
